# baseline (speedup 1.0000x reference)
import jax
import jax.numpy as jnp
from jax.experimental import pallas as pl
from jax.experimental.pallas import tpu as pltpu

_SCORE_THRESHOLD = 0.05
_MAX_DETECTIONS = 300
_NMS_THRESHOLD = 0.5

_ROWS = 160
_LANES = 128
_P = _ROWS * _LANES
_G = _ROWS // 8


def _nms_body(boxes_ref, cls_ref, out_ref):
    X1 = boxes_ref[0, 0]
    Y1 = boxes_ref[0, 1]
    X2 = boxes_ref[0, 2]
    Y2 = boxes_ref[0, 3]
    AREA = (X2 - X1) * (Y2 - Y1)

    C = cls_ref.shape[1]
    best = cls_ref[0, 0]
    labf = jnp.zeros((_ROWS, _LANES), jnp.float32)
    for c in range(1, C):
        v = cls_ref[0, c]
        gt = v > best
        best = jnp.where(gt, v, best)
        labf = jnp.where(gt, jnp.float32(c), labf)

    NEG = jnp.float32(-jnp.inf)
    work0 = jnp.where(best > _SCORE_THRESHOLD, best, NEG)

    I = (jax.lax.broadcasted_iota(jnp.int32, (_ROWS, _LANES), 0) * _LANES
         + jax.lax.broadcasted_iota(jnp.int32, (_ROWS, _LANES), 1))
    lane = jax.lax.broadcasted_iota(jnp.int32, (1, _LANES), 1)

    def grp(a):
        return [a[8 * j:8 * j + 8] for j in range(_G)]

    X1g, Y1g, X2g, Y2g, Lg, Ig = (grp(a) for a in (X1, Y1, X2, Y2, labf, I))

    def body(i, work):
        wg = grp(work)
        items = [(wg[j], Ig[j], X1g[j], Y1g[j], X2g[j], Y2g[j], Lg[j])
                 for j in range(_G)]
        while len(items) > 1:
            nxt = []
            for k in range(0, len(items) - 1, 2):
                a, b = items[k], items[k + 1]
                take = b[0] > a[0]
                nxt.append(tuple(jnp.where(take, y, x) for x, y in zip(a, b)))
            if len(items) % 2:
                nxt.append(items[-1])
            items = nxt
        t = items[0]
        for axis, size in ((0, 8), (1, _LANES)):
            s = 1
            while s < size:
                r = tuple(pltpu.roll(x, s, axis) for x in t)
                take = (r[0] > t[0]) | ((r[0] == t[0]) & (r[1] < t[1]))
                t = tuple(jnp.where(take, y, x) for x, y in zip(t, r))
                s *= 2
        m, _, x1b, y1b, x2b, y2b, lb = (x[0:1, :] for x in t)
        has = m > NEG
        ab = (x2b - x1b) * (y2b - y1b)

        ix1 = jnp.maximum(X1, x1b)
        iy1 = jnp.maximum(Y1, y1b)
        ix2 = jnp.minimum(X2, x2b)
        iy2 = jnp.minimum(Y2, y2b)
        iw = jnp.maximum(ix2 - ix1, 0.0)
        ih = jnp.maximum(iy2 - iy1, 0.0)
        inter = iw * ih
        union = AREA + ab - inter
        iou = inter / jnp.maximum(union, 1e-8)
        sup = (iou > _NMS_THRESHOLD) & has
        work = jnp.where(sup, NEG, work)

        row = jnp.full((1, _LANES), -1.0, jnp.float32)
        for j, v in enumerate((x1b, y1b, x2b, y2b, m, lb)):
            row = jnp.where(lane == j, v, row)
        row = jnp.where(has, row, jnp.float32(-1.0))
        out_ref[0, pl.ds(i, 1), :] = row
        return work

    jax.lax.fori_loop(0, _MAX_DETECTIONS, body, work0, unroll=2)


def kernel(boxes, classification):
    B, N, C = classification.shape
    bt = jnp.transpose(boxes, (0, 2, 1))
    bt = jnp.pad(bt, ((0, 0), (0, 0), (0, _P - N)))
    bt = bt.reshape(B, 4, _ROWS, _LANES)
    ct = jnp.transpose(classification, (0, 2, 1))
    ct = jnp.pad(ct, ((0, 0), (0, 0), (0, _P - N)), constant_values=-1.0)
    ct = ct.reshape(B, C, _ROWS, _LANES)

    out = pl.pallas_call(
        _nms_body,
        grid=(B,),
        in_specs=[
            pl.BlockSpec((1, 4, _ROWS, _LANES), lambda b: (b, 0, 0, 0)),
            pl.BlockSpec((1, C, _ROWS, _LANES), lambda b: (b, 0, 0, 0)),
        ],
        out_specs=pl.BlockSpec((1, 304, _LANES), lambda b: (b, 0, 0)),
        out_shape=jax.ShapeDtypeStruct((B, 304, _LANES), jnp.float32),
        compiler_params=pltpu.CompilerParams(
            dimension_semantics=("arbitrary",),
        ),
    )(bt, ct)

    out_boxes = out[:, :_MAX_DETECTIONS, 0:4]
    out_scores = out[:, :_MAX_DETECTIONS, 4]
    out_labels = out[:, :_MAX_DETECTIONS, 5].astype(jnp.int32)
    return out_boxes, out_scores, out_labels


# speedup vs baseline: 1.2193x; 1.2193x over previous
import jax
import jax.numpy as jnp
from jax.experimental import pallas as pl
from jax.experimental.pallas import tpu as pltpu

_SCORE_THRESHOLD = 0.05
_MAX_DETECTIONS = 300
_NMS_THRESHOLD = 0.5

_ROWS = 160
_LANES = 128
_P = _ROWS * _LANES


def _nms_body(boxes_ref, cls_ref, out_ref):
    B = boxes_ref.shape[0]
    C = cls_ref.shape[1]
    NEG = jnp.float32(-jnp.inf)

    I = (jax.lax.broadcasted_iota(jnp.int32, (_ROWS, _LANES), 0) * _LANES
         + jax.lax.broadcasted_iota(jnp.int32, (_ROWS, _LANES), 1))
    lane = jax.lax.broadcasted_iota(jnp.int32, (1, _LANES), 1)

    X1, Y1, X2, Y2, AREA, LABF, WORK0 = [], [], [], [], [], [], []
    for b in range(B):
        x1 = boxes_ref[b, 0]
        y1 = boxes_ref[b, 1]
        x2 = boxes_ref[b, 2]
        y2 = boxes_ref[b, 3]
        X1.append(x1)
        Y1.append(y1)
        X2.append(x2)
        Y2.append(y2)
        AREA.append((x2 - x1) * (y2 - y1))
        best = cls_ref[b, 0]
        labf = jnp.zeros((_ROWS, _LANES), jnp.float32)
        for c in range(1, C):
            v = cls_ref[b, c]
            gt = v > best
            best = jnp.where(gt, v, best)
            labf = jnp.where(gt, jnp.float32(c), labf)
        LABF.append(labf)
        WORK0.append(jnp.where(best > _SCORE_THRESHOLD, best, NEG))

    def body(i, works):
        new_works = []
        for b in range(B):
            work = works[b]
            m = jnp.max(work)
            has = m > NEG
            idx = jnp.min(jnp.where(work == m, I, jnp.int32(1 << 30)))
            flag = I == idx

            def pick(a):
                return jnp.sum(jnp.where(flag, a, 0.0))

            x1b = pick(X1[b])
            y1b = pick(Y1[b])
            x2b = pick(X2[b])
            y2b = pick(Y2[b])
            lb = pick(LABF[b])
            ab = (x2b - x1b) * (y2b - y1b)

            ix1 = jnp.maximum(X1[b], x1b)
            iy1 = jnp.maximum(Y1[b], y1b)
            ix2 = jnp.minimum(X2[b], x2b)
            iy2 = jnp.minimum(Y2[b], y2b)
            iw = jnp.maximum(ix2 - ix1, 0.0)
            ih = jnp.maximum(iy2 - iy1, 0.0)
            inter = iw * ih
            union = AREA[b] + ab - inter
            iou = inter / jnp.maximum(union, 1e-8)
            sup = (iou > _NMS_THRESHOLD) & has
            new_works.append(jnp.where(sup, NEG, work))

            row = jnp.full((1, _LANES), -1.0, jnp.float32)
            for j, v in enumerate((x1b, y1b, x2b, y2b, m, lb)):
                row = jnp.where(lane == j, v, row)
            row = jnp.where(has, row, jnp.float32(-1.0))
            out_ref[b, pl.ds(i, 1), :] = row
        return tuple(new_works)

    jax.lax.fori_loop(0, _MAX_DETECTIONS, body, tuple(WORK0))


def kernel(boxes, classification):
    B, N, C = classification.shape
    bt = jnp.transpose(boxes, (0, 2, 1))
    bt = jnp.pad(bt, ((0, 0), (0, 0), (0, _P - N)))
    bt = bt.reshape(B, 4, _ROWS, _LANES)
    ct = jnp.transpose(classification, (0, 2, 1))
    ct = jnp.pad(ct, ((0, 0), (0, 0), (0, _P - N)), constant_values=-1.0)
    ct = ct.reshape(B, C, _ROWS, _LANES)

    out = pl.pallas_call(
        _nms_body,
        in_specs=[
            pl.BlockSpec((B, 4, _ROWS, _LANES), lambda: (0, 0, 0, 0)),
            pl.BlockSpec((B, C, _ROWS, _LANES), lambda: (0, 0, 0, 0)),
        ],
        out_specs=pl.BlockSpec((B, 304, _LANES), lambda: (0, 0, 0)),
        out_shape=jax.ShapeDtypeStruct((B, 304, _LANES), jnp.float32),
    )(bt, ct)

    out_boxes = out[:, :_MAX_DETECTIONS, 0:4]
    out_scores = out[:, :_MAX_DETECTIONS, 4]
    out_labels = out[:, :_MAX_DETECTIONS, 5].astype(jnp.int32)
    return out_boxes, out_scores, out_labels


# speedup vs baseline: 1.2655x; 1.0379x over previous
import jax
import jax.numpy as jnp
from jax.experimental import pallas as pl
from jax.experimental.pallas import tpu as pltpu

_SCORE_THRESHOLD = 0.05
_MAX_DETECTIONS = 300
_NMS_THRESHOLD = 0.5

_ROWS = 160
_LANES = 128
_P = _ROWS * _LANES


def _nms_body(boxes_ref, cls_ref, out_ref, work_ref, lab_ref, area_ref):
    B = boxes_ref.shape[0]
    C = cls_ref.shape[1]
    NEG = jnp.float32(-jnp.inf)

    I = (jax.lax.broadcasted_iota(jnp.int32, (_ROWS, _LANES), 0) * _LANES
         + jax.lax.broadcasted_iota(jnp.int32, (_ROWS, _LANES), 1))
    lane = jax.lax.broadcasted_iota(jnp.int32, (1, _LANES), 1)

    for b in range(B):
        x1 = boxes_ref[b, 0]
        y1 = boxes_ref[b, 1]
        x2 = boxes_ref[b, 2]
        y2 = boxes_ref[b, 3]
        area_ref[b] = (x2 - x1) * (y2 - y1)
        best = cls_ref[b, 0]
        labf = jnp.zeros((_ROWS, _LANES), jnp.float32)
        for c in range(1, C):
            v = cls_ref[b, c]
            gt = v > best
            best = jnp.where(gt, v, best)
            labf = jnp.where(gt, jnp.float32(c), labf)
        lab_ref[b] = labf
        work_ref[b] = jnp.where(best > _SCORE_THRESHOLD, best, NEG)

    def body(i, carry):
        for b in range(B):
            work = work_ref[b]
            m = jnp.max(work)
            has = m > NEG
            idx = jnp.min(jnp.where(work == m, I, jnp.int32(1 << 30)))
            flag = I == idx

            def pick(a):
                return jnp.sum(jnp.where(flag, a, 0.0))

            X1 = boxes_ref[b, 0]
            Y1 = boxes_ref[b, 1]
            X2 = boxes_ref[b, 2]
            Y2 = boxes_ref[b, 3]
            x1b = pick(X1)
            y1b = pick(Y1)
            x2b = pick(X2)
            y2b = pick(Y2)
            lb = pick(lab_ref[b])
            ab = (x2b - x1b) * (y2b - y1b)

            ix1 = jnp.maximum(X1, x1b)
            iy1 = jnp.maximum(Y1, y1b)
            ix2 = jnp.minimum(X2, x2b)
            iy2 = jnp.minimum(Y2, y2b)
            iw = jnp.maximum(ix2 - ix1, 0.0)
            ih = jnp.maximum(iy2 - iy1, 0.0)
            inter = iw * ih
            union = area_ref[b] + ab - inter
            iou = inter / jnp.maximum(union, 1e-8)
            sup = (iou > _NMS_THRESHOLD) & has
            work_ref[b] = jnp.where(sup, NEG, work)

            row = jnp.full((1, _LANES), -1.0, jnp.float32)
            for j, v in enumerate((x1b, y1b, x2b, y2b, m, lb)):
                row = jnp.where(lane == j, v, row)
            row = jnp.where(has, row, jnp.float32(-1.0))
            out_ref[b, pl.ds(i, 1), :] = row
        return carry

    jax.lax.fori_loop(0, _MAX_DETECTIONS, body, 0)


def kernel(boxes, classification):
    B, N, C = classification.shape
    bt = jnp.transpose(boxes, (0, 2, 1))
    bt = jnp.pad(bt, ((0, 0), (0, 0), (0, _P - N)))
    bt = bt.reshape(B, 4, _ROWS, _LANES)
    ct = jnp.transpose(classification, (0, 2, 1))
    ct = jnp.pad(ct, ((0, 0), (0, 0), (0, _P - N)), constant_values=-1.0)
    ct = ct.reshape(B, C, _ROWS, _LANES)

    out = pl.pallas_call(
        _nms_body,
        in_specs=[
            pl.BlockSpec((B, 4, _ROWS, _LANES), lambda: (0, 0, 0, 0)),
            pl.BlockSpec((B, C, _ROWS, _LANES), lambda: (0, 0, 0, 0)),
        ],
        out_specs=pl.BlockSpec((B, 304, _LANES), lambda: (0, 0, 0)),
        out_shape=jax.ShapeDtypeStruct((B, 304, _LANES), jnp.float32),
        scratch_shapes=[
            pltpu.VMEM((B, _ROWS, _LANES), jnp.float32),
            pltpu.VMEM((B, _ROWS, _LANES), jnp.float32),
            pltpu.VMEM((B, _ROWS, _LANES), jnp.float32),
        ],
    )(bt, ct)

    out_boxes = out[:, :_MAX_DETECTIONS, 0:4]
    out_scores = out[:, :_MAX_DETECTIONS, 4]
    out_labels = out[:, :_MAX_DETECTIONS, 5].astype(jnp.int32)
    return out_boxes, out_scores, out_labels


# speedup vs baseline: 2.6121x; 2.0640x over previous
import jax
import jax.numpy as jnp
from jax.experimental import pallas as pl
from jax.experimental.pallas import tpu as pltpu

_SCORE_THRESHOLD = 0.05
_MAX_DETECTIONS = 300
_NMS_THRESHOLD = 0.5

_ROWS = 160
_LANES = 128
_P = _ROWS * _LANES


def _nms_body(boxes_ref, cls_ref, out_ref, work_ref, lab_ref, area_ref):
    B = boxes_ref.shape[0]
    C = cls_ref.shape[1]
    NEG = jnp.float32(-jnp.inf)

    I = (jax.lax.broadcasted_iota(jnp.int32, (_ROWS, _LANES), 0) * _LANES
         + jax.lax.broadcasted_iota(jnp.int32, (_ROWS, _LANES), 1))[None]
    lane = jax.lax.broadcasted_iota(jnp.int32, (1, 1, _LANES), 2)

    x1 = boxes_ref[:, 0]
    x2 = boxes_ref[:, 2]
    y1 = boxes_ref[:, 1]
    y2 = boxes_ref[:, 3]
    area_ref[...] = (x2 - x1) * (y2 - y1)
    best = cls_ref[:, 0]
    labf = jnp.zeros((B, _ROWS, _LANES), jnp.float32)
    for c in range(1, C):
        v = cls_ref[:, c]
        gt = v > best
        best = jnp.where(gt, v, best)
        labf = jnp.where(gt, jnp.float32(c), labf)
    lab_ref[...] = labf
    work_ref[...] = jnp.where(best > _SCORE_THRESHOLD, best, NEG)

    def body(i, carry):
        work = work_ref[...]
        m = jnp.max(work, axis=(1, 2), keepdims=True)
        has = m > NEG
        idx = jnp.min(jnp.where(work == m, I, jnp.int32(1 << 30)),
                      axis=(1, 2), keepdims=True)
        flag = I == idx

        def pick(a):
            return jnp.sum(jnp.where(flag, a, 0.0), axis=(1, 2), keepdims=True)

        X1 = boxes_ref[:, 0]
        Y1 = boxes_ref[:, 1]
        X2 = boxes_ref[:, 2]
        Y2 = boxes_ref[:, 3]
        x1b = pick(X1)
        y1b = pick(Y1)
        x2b = pick(X2)
        y2b = pick(Y2)
        lb = pick(lab_ref[...])
        ab = (x2b - x1b) * (y2b - y1b)

        ix1 = jnp.maximum(X1, x1b)
        iy1 = jnp.maximum(Y1, y1b)
        ix2 = jnp.minimum(X2, x2b)
        iy2 = jnp.minimum(Y2, y2b)
        iw = jnp.maximum(ix2 - ix1, 0.0)
        ih = jnp.maximum(iy2 - iy1, 0.0)
        inter = iw * ih
        union = area_ref[...] + ab - inter
        iou = inter / jnp.maximum(union, 1e-8)
        sup = (iou > _NMS_THRESHOLD) & has
        work_ref[...] = jnp.where(sup, NEG, work)

        row = jnp.full((B, 1, _LANES), -1.0, jnp.float32)
        for j, v in enumerate((x1b, y1b, x2b, y2b, m, lb)):
            row = jnp.where(lane == j, v, row)
        row = jnp.where(has, row, jnp.float32(-1.0))
        out_ref[:, pl.ds(i, 1), :] = row
        return carry

    jax.lax.fori_loop(0, _MAX_DETECTIONS, body, 0)


def kernel(boxes, classification):
    B, N, C = classification.shape
    bt = jnp.transpose(boxes, (0, 2, 1))
    bt = jnp.pad(bt, ((0, 0), (0, 0), (0, _P - N)))
    bt = bt.reshape(B, 4, _ROWS, _LANES)
    ct = jnp.transpose(classification, (0, 2, 1))
    ct = jnp.pad(ct, ((0, 0), (0, 0), (0, _P - N)), constant_values=-1.0)
    ct = ct.reshape(B, C, _ROWS, _LANES)

    out = pl.pallas_call(
        _nms_body,
        in_specs=[
            pl.BlockSpec((B, 4, _ROWS, _LANES), lambda: (0, 0, 0, 0)),
            pl.BlockSpec((B, C, _ROWS, _LANES), lambda: (0, 0, 0, 0)),
        ],
        out_specs=pl.BlockSpec((B, 304, _LANES), lambda: (0, 0, 0)),
        out_shape=jax.ShapeDtypeStruct((B, 304, _LANES), jnp.float32),
        scratch_shapes=[
            pltpu.VMEM((B, _ROWS, _LANES), jnp.float32),
            pltpu.VMEM((B, _ROWS, _LANES), jnp.float32),
            pltpu.VMEM((B, _ROWS, _LANES), jnp.float32),
        ],
    )(bt, ct)

    out_boxes = out[:, :_MAX_DETECTIONS, 0:4]
    out_scores = out[:, :_MAX_DETECTIONS, 4]
    out_labels = out[:, :_MAX_DETECTIONS, 5].astype(jnp.int32)
    return out_boxes, out_scores, out_labels


# speedup vs baseline: 2.7169x; 1.0401x over previous
import jax
import jax.numpy as jnp
from jax.experimental import pallas as pl
from jax.experimental.pallas import tpu as pltpu

_SCORE_THRESHOLD = 0.05
_MAX_DETECTIONS = 300
_NMS_THRESHOLD = 0.5

_ROWS = 160
_LANES = 128
_P = _ROWS * _LANES


def _nms_body(boxes_ref, cls_ref, out_ref, work_ref, lab_ref, area_ref):
    B = boxes_ref.shape[0]
    C = cls_ref.shape[1]
    NEG = jnp.float32(-jnp.inf)

    I = (jax.lax.broadcasted_iota(jnp.int32, (_ROWS, _LANES), 0) * _LANES
         + jax.lax.broadcasted_iota(jnp.int32, (_ROWS, _LANES), 1))[None]
    lane = jax.lax.broadcasted_iota(jnp.int32, (1, 1, _LANES), 2)

    x1 = boxes_ref[:, 0]
    x2 = boxes_ref[:, 2]
    y1 = boxes_ref[:, 1]
    y2 = boxes_ref[:, 3]
    area_ref[...] = (x2 - x1) * (y2 - y1)
    best = cls_ref[:, 0]
    labf = jnp.zeros((B, _ROWS, _LANES), jnp.float32)
    for c in range(1, C):
        v = cls_ref[:, c]
        gt = v > best
        best = jnp.where(gt, v, best)
        labf = jnp.where(gt, jnp.float32(c), labf)
    lab_ref[...] = labf
    work_ref[...] = jnp.where(best > _SCORE_THRESHOLD, best, NEG)

    def body(i, carry):
        work = work_ref[...]
        m = jnp.max(work, axis=(1, 2), keepdims=True)
        has = m > NEG
        idx = jnp.min(jnp.where(work == m, I, jnp.int32(1 << 30)),
                      axis=(1, 2), keepdims=True)
        flag = I == idx

        def pick(a):
            return jnp.sum(jnp.where(flag, a, 0.0), axis=(1, 2), keepdims=True)

        X1 = boxes_ref[:, 0]
        Y1 = boxes_ref[:, 1]
        X2 = boxes_ref[:, 2]
        Y2 = boxes_ref[:, 3]
        x1b = pick(X1)
        y1b = pick(Y1)
        x2b = pick(X2)
        y2b = pick(Y2)
        lb = pick(lab_ref[...])
        ab = (x2b - x1b) * (y2b - y1b)

        ix1 = jnp.maximum(X1, x1b)
        iy1 = jnp.maximum(Y1, y1b)
        ix2 = jnp.minimum(X2, x2b)
        iy2 = jnp.minimum(Y2, y2b)
        iw = jnp.maximum(ix2 - ix1, 0.0)
        ih = jnp.maximum(iy2 - iy1, 0.0)
        inter = iw * ih
        union = area_ref[...] + ab - inter
        iou = inter / jnp.maximum(union, 1e-8)
        sup = (iou > _NMS_THRESHOLD) & has
        work_ref[...] = jnp.where(sup, NEG, work)

        row = jnp.full((B, 1, _LANES), -1.0, jnp.float32)
        for j, v in enumerate((x1b, y1b, x2b, y2b, m, lb)):
            row = jnp.where(lane == j, v, row)
        row = jnp.where(has, row, jnp.float32(-1.0))
        out_ref[:, pl.ds(i, 1), :] = row
        return carry

    jax.lax.fori_loop(0, _MAX_DETECTIONS, body, 0, unroll=2)


def kernel(boxes, classification):
    B, N, C = classification.shape
    bt = jnp.transpose(boxes, (0, 2, 1))
    bt = jnp.pad(bt, ((0, 0), (0, 0), (0, _P - N)))
    bt = bt.reshape(B, 4, _ROWS, _LANES)
    ct = jnp.transpose(classification, (0, 2, 1))
    ct = jnp.pad(ct, ((0, 0), (0, 0), (0, _P - N)), constant_values=-1.0)
    ct = ct.reshape(B, C, _ROWS, _LANES)

    out = pl.pallas_call(
        _nms_body,
        in_specs=[
            pl.BlockSpec((B, 4, _ROWS, _LANES), lambda: (0, 0, 0, 0)),
            pl.BlockSpec((B, C, _ROWS, _LANES), lambda: (0, 0, 0, 0)),
        ],
        out_specs=pl.BlockSpec((B, 304, _LANES), lambda: (0, 0, 0)),
        out_shape=jax.ShapeDtypeStruct((B, 304, _LANES), jnp.float32),
        scratch_shapes=[
            pltpu.VMEM((B, _ROWS, _LANES), jnp.float32),
            pltpu.VMEM((B, _ROWS, _LANES), jnp.float32),
            pltpu.VMEM((B, _ROWS, _LANES), jnp.float32),
        ],
    )(bt, ct)

    out_boxes = out[:, :_MAX_DETECTIONS, 0:4]
    out_scores = out[:, :_MAX_DETECTIONS, 4]
    out_labels = out[:, :_MAX_DETECTIONS, 5].astype(jnp.int32)
    return out_boxes, out_scores, out_labels
